# Initial kernel scaffold; baseline (speedup 1.0000x reference)
import jax
import jax.numpy as jnp
from jax.experimental import pallas as pl


def kernel(x, feature_mask):
    raise NotImplementedError("write your pallas kernel here")


# speedup vs baseline: 2.2460x; 2.2460x over previous
import jax
import jax.numpy as jnp
from jax.experimental import pallas as pl


def _tc_body(x_ref, onehot_ref, out_ref):
    out_ref[...] = jnp.dot(
        x_ref[...], onehot_ref[...], preferred_element_type=jnp.float32
    )


def kernel(x, feature_mask):
    B, S, F = x.shape
    K = feature_mask.shape[0]
    R = B * S
    x2 = x.reshape(R, F)
    onehot = (
        feature_mask[None, :] == jax.lax.iota(jnp.int32, F)[:, None]
    ).astype(jnp.float32)

    BLK = 512
    out = pl.pallas_call(
        _tc_body,
        grid=(R // BLK,),
        in_specs=[
            pl.BlockSpec((BLK, F), lambda i: (i, 0)),
            pl.BlockSpec((F, K), lambda i: (0, 0)),
        ],
        out_specs=pl.BlockSpec((BLK, K), lambda i: (i, 0)),
        out_shape=jax.ShapeDtypeStruct((R, K), jnp.float32),
    )(x2, onehot)
    return out.reshape(B, S, K)
